# baseline (speedup 1.0000x reference)
import jax
import jax.numpy as jnp
from jax import lax
from jax.experimental import pallas as pl
from jax.experimental.pallas import tpu as pltpu
from jax.experimental.pallas import tpu_sc as plsc

_N = 100000
_E = 1600000
_H = 32
_HH = 16
_K = 5
_IN_SZ = 1000
_OUT = 10

_NSC = 2
_NTILE = 16
_HALF = _N // _NSC
_G = 128
_NGRP = 12800
_EP = _NGRP * _G
_GPT = _NGRP // _NTILE
_B = 2
_BE = _B * _G
_BNC = 104
_ROWS_PT = 3128
_SROWS = _NTILE * _ROWS_PT
_DUMP = _HALF + 5

_R = 2000
_NBLK = _N // _R


def _make_sc_prop(width, histogram, nslab=1):
    if width == 1:
        rows_s, acc_s, bnc_s, out_s = (_BE,), (_SROWS,), (_BNC,), (_N,)
    else:
        rows_s = (_BE, width)
        acc_s = (_SROWS, width)
        bnc_s = (_BNC, width)
        out_s = (_N, width)
    mesh = plsc.VectorSubcoreMesh(core_axis_name="c", subcore_axis_name="s")
    scratch = [
        pltpu.VMEM((_BE,), jnp.int32),
        pltpu.VMEM((_BE,), jnp.int32),
        pltpu.VMEM((_BE,), jnp.int32),
        pltpu.VMEM((_BE,), jnp.int32),
        pltpu.VMEM(rows_s, jnp.float32),
        pltpu.VMEM(rows_s, jnp.float32),
        pltpu.VMEM(bnc_s, jnp.float32),
        pltpu.VMEM_SHARED(acc_s, jnp.float32),
        pltpu.SemaphoreType.DMA,
        pltpu.SemaphoreType.DMA,
        pltpu.SemaphoreType.DMA,
        pltpu.SemaphoreType.DMA,
    ]

    def body(gidx_hbm, sidx_hbm, *rest):
        w_hbms = rest[:nslab]
        zeros_hbm = rest[nslab]
        out_hbms = rest[nslab + 1:2 * nslab + 1]
        (rowi, coli, tgti0, tgti1, rows0, rows1, wb, accum,
         sg0, sg1, ss0, ss1) = rest[2 * nslab + 1:]
        c = lax.axis_index("c")
        s = lax.axis_index("s")
        base = c * _HALF
        lo = s * _ROWS_PT
        tail = _HALF - (_NTILE - 1) * _ROWS_PT

        pltpu.sync_copy(zeros_hbm, wb)
        if histogram:
            pltpu.sync_copy(w_hbms[0], rows0)

        def _chunks(total):
            offs = []
            o = 0
            while o < total:
                offs.append((o, min(_BNC, total - o)))
                o += _BNC
            return offs

        for slab in range(nslab):
            w_hbm = w_hbms[slab]
            out_hbm = out_hbms[slab]

            for off, sz in _chunks(_ROWS_PT):
                pltpu.sync_copy(wb.at[pl.ds(0, sz)],
                                accum.at[pl.ds(lo + off, sz)])
            plsc.subcore_barrier()

            nb = _GPT // _B

            def _compute_tgt(e0, tgti):
                pltpu.sync_copy(sidx_hbm.at[pl.ds(e0, _BE)], coli)
                for j in range(_BE // 16):
                    v = coli[pl.ds(j * 16, 16)]
                    t0 = v - base
                    ok = (t0 >= 0) & (t0 < _HALF)
                    tgti[pl.ds(j * 16, 16)] = jnp.where(ok, t0, _DUMP)

            if histogram:
                def hstep(bt, carry):
                    e0 = (s * _GPT + bt * _B) * _G
                    _compute_tgt(e0, tgti0)
                    pltpu.sync_copy(rows0, accum.at[tgti0], add=True)
                    return carry

                lax.fori_loop(0, nb, hstep, 0)
            else:
                bufs = ((tgti0, rows0, sg0, ss0), (tgti1, rows1, sg1, ss1))

                def _half(i, k, par):
                    tg, rw, sg, ss = bufs[par]
                    tgp, rwp, sgp, ssp = bufs[1 - par]
                    e0 = (s * _GPT + i * _B) * _G

                    @pl.when(k > 0)
                    def _():
                        pltpu.make_async_copy(
                            rw, accum.at[tg], ss).wait()

                    _compute_tgt(e0, tg)

                    @pl.when(i > 0)
                    def _():
                        pltpu.make_async_copy(
                            w_hbm.at[rowi], rwp, sgp).wait()
                        pltpu.async_copy(
                            rwp, accum.at[tgp], ssp, add=True)

                    pltpu.sync_copy(gidx_hbm.at[pl.ds(e0, _BE)], rowi)
                    pltpu.async_copy(w_hbm.at[rowi], rw, sg)

                def pstep(k, carry):
                    _half(2 * k, k, 0)
                    _half(2 * k + 1, k, 1)
                    return carry

                lax.fori_loop(0, nb // 2, pstep, 0)
                par_last = (nb - 1) % 2
                tg, rw, sg, ss = bufs[par_last]
                tgp, rwp, sgp, ssp = bufs[1 - par_last]
                pltpu.make_async_copy(w_hbm.at[rowi], rw, sg).wait()
                pltpu.async_copy(rw, accum.at[tg], ss, add=True)
                pltpu.make_async_copy(rwp, accum.at[tgp], ssp).wait()
                pltpu.make_async_copy(rw, accum.at[tg], ss).wait()
            plsc.subcore_barrier()

            @pl.when(s < _NTILE - 1)
            def _():
                for off, sz in _chunks(_ROWS_PT):
                    pltpu.sync_copy(accum.at[pl.ds(lo + off, sz)],
                                    wb.at[pl.ds(0, sz)])
                    pltpu.sync_copy(wb.at[pl.ds(0, sz)],
                                    out_hbm.at[pl.ds(base + lo + off, sz)])

            @pl.when(s == _NTILE - 1)
            def _():
                for off, sz in _chunks(tail):
                    pltpu.sync_copy(accum.at[pl.ds(lo + off, sz)],
                                    wb.at[pl.ds(0, sz)])
                    pltpu.sync_copy(wb.at[pl.ds(0, sz)],
                                    out_hbm.at[pl.ds(base + lo + off, sz)])

            if slab + 1 < nslab:
                pltpu.sync_copy(zeros_hbm, wb)

    if nslab == 1:
        out_type = jax.ShapeDtypeStruct(out_s, jnp.float32)
    else:
        out_type = [jax.ShapeDtypeStruct(out_s, jnp.float32)] * 2
    return pl.kernel(
        body,
        out_type=out_type,
        mesh=mesh,
        scratch_types=scratch,
        compiler_params=pltpu.CompilerParams(use_tc_tiling_on_sc=False),
    )


def _row_spec(w):
    return pl.BlockSpec((_R, w), lambda i: (i, 0))


def _full_spec(shape):
    return pl.BlockSpec(shape, lambda i: tuple(0 for _ in shape))


def _tc_rsqrt(deg):
    def body(d_ref, o_ref):
        d = d_ref[...]
        o_ref[...] = jnp.where(d > 0, lax.rsqrt(jnp.where(d > 0, d, 1.0)), 0.0)

    return pl.pallas_call(
        body,
        grid=(_NBLK,),
        in_specs=[_row_spec(1)],
        out_specs=_row_spec(1),
        out_shape=jax.ShapeDtypeStruct((_N, 1), jnp.float32),
    )(deg)


def _tc_init(x, dis2, W1r, b):
    def body(x_ref, d_ref, w_ref, b_ref, out_ref, h_ref, wout_ref):
        xv = x_ref[...]
        out_ref[...] = xv * w_ref[...] + b_ref[...]
        hv = jnp.broadcast_to(xv, (_R, _H))
        h_ref[...] = hv
        wout_ref[...] = d_ref[...] * hv

    return pl.pallas_call(
        body,
        grid=(_NBLK,),
        in_specs=[_row_spec(1), _row_spec(1), _full_spec((1, _H)),
                  _full_spec((1, _H))],
        out_specs=[_row_spec(_H)] * 3,
        out_shape=[jax.ShapeDtypeStruct((_N, _H), jnp.float32)] * 3,
    )(x, dis2, W1r, b)


def _tc_step(acc, prev2, prev1, out_in, dis2, Wk, alpha, tend, W0n, b0n):
    def body(a_ref, p2_ref, p1_ref, out_ref, d_ref, w_ref, al_ref, te_ref,
             w0_ref, b0_ref, np2_ref, np1_ref, nw_ref, nout_ref):
        d = d_ref[...]
        tx = al_ref[0, 0] * (d * a_ref[...]) - p2_ref[...]
        o1 = out_ref[...] + jnp.dot(tx, w_ref[...],
                                    preferred_element_type=jnp.float32)
        te = te_ref[0, 0]
        h2 = jnp.maximum(o1, 0.0)
        o2 = jnp.dot(h2, w0_ref[...],
                     preferred_element_type=jnp.float32) + b0_ref[...]
        np2_ref[...] = (1.0 - te) * p1_ref[...]
        np1 = te * h2 + (1.0 - te) * tx
        np1_ref[...] = np1
        nw_ref[...] = d * np1
        nout_ref[...] = te * o2 + (1.0 - te) * o1

    return pl.pallas_call(
        body,
        grid=(_NBLK,),
        in_specs=[_row_spec(_H), _row_spec(_H), _row_spec(_H), _row_spec(_H),
                  _row_spec(1), _full_spec((_H, _H)), _full_spec((1, 1)),
                  _full_spec((1, 1)), _full_spec((_H, _H)),
                  _full_spec((1, _H))],
        out_specs=[_row_spec(_H)] * 4,
        out_shape=[jax.ShapeDtypeStruct((_N, _H), jnp.float32)] * 4,
    )(acc, prev2, prev1, out_in, dis2, Wk, alpha, tend, W0n, b0n)


def _tc_final(hm, Wl, bl2):
    kb = 3200
    nk = (_IN_SZ * _H) // kb
    ng = _N // _IN_SZ

    def body(h_ref, w_ref, b_ref, o_ref):
        @pl.when(pl.program_id(0) == 0)
        def _():
            o_ref[...] = jnp.zeros((ng, _OUT), jnp.float32) + b_ref[...]

        o_ref[...] += jnp.dot(h_ref[...], w_ref[...],
                              preferred_element_type=jnp.float32)

    return pl.pallas_call(
        body,
        grid=(nk,),
        in_specs=[pl.BlockSpec((ng, kb), lambda i: (0, i)),
                  pl.BlockSpec((kb, _OUT), lambda i: (i, 0)),
                  pl.BlockSpec((1, _OUT), lambda i: (0, 0))],
        out_specs=pl.BlockSpec((ng, _OUT), lambda i: (0, 0)),
        out_shape=jax.ShapeDtypeStruct((ng, _OUT), jnp.float32),
    )(hm, Wl, bl2)


_sc_hist = _make_sc_prop(1, histogram=True)
_sc_prop = _make_sc_prop(_H, histogram=False)


def kernel(x, edge_index, batch, W1, b1, W2, b2, W3, b3, Wl, bl):
    row = edge_index[0]
    col = edge_index[1]
    pad = _EP - _E
    rowg = jnp.concatenate([row, jnp.zeros((pad,), jnp.int32)])
    rowh = jnp.concatenate([row, jnp.full((pad,), _N, jnp.int32)])
    cols = jnp.concatenate([col, jnp.full((pad,), _N, jnp.int32)])
    z1 = jnp.zeros((_BNC,), jnp.float32)
    zH = jnp.zeros((_BNC, _H), jnp.float32)
    onesg = jnp.ones((_BE,), jnp.float32)

    deg = _sc_hist(rowg, rowh, onesg, z1)
    dis2 = _tc_rsqrt(deg.reshape(_N, 1))

    def padW1(k):
        return jnp.zeros((_H, _H), jnp.float32).at[0].set(W1[k, 0])

    Wks = jnp.stack([padW1(1), padW1(2), padW1(3), padW1(4),
                     W2[1], W2[2], W2[3], W2[4],
                     W3[1], W3[2], W3[3], W3[4]])
    alphas = jnp.tile(jnp.array([-1.0, -2.0, -2.0, -2.0], jnp.float32),
                      3).reshape(12, 1, 1)
    tends = jnp.array([0, 0, 0, 1, 0, 0, 0, 1, 0, 0, 0, 0],
                      jnp.float32).reshape(12, 1, 1)
    zW0 = jnp.zeros((_H, _H), jnp.float32)
    zb0 = jnp.zeros((1, _H), jnp.float32)
    W0s = jnp.stack([zW0, zW0, zW0, W2[0], zW0, zW0, zW0, W3[0],
                     zW0, zW0, zW0, zW0])
    b0s = jnp.stack([zb0, zb0, zb0, b2.reshape(1, _H), zb0, zb0, zb0,
                     b3.reshape(1, _H), zb0, zb0, zb0, zb0])

    out0, h0, w0 = _tc_init(x, dis2, W1[0].reshape(1, _H),
                            b1.reshape(1, _H))

    def step(carry, xs):
        prev2, prev1, w, out = carry
        Wk, alpha, tend, W0n, b0n = xs
        acc = _sc_prop(rowg, cols, w, zH)
        np2, np1, nw, nout = _tc_step(acc, prev2, prev1, out, dis2,
                                      Wk, alpha, tend, W0n, b0n)
        return (np2, np1, nw, nout), 0.0

    init = (jnp.zeros((_N, _H), jnp.float32), h0, w0, out0)
    (_, _, _, out), _ = lax.scan(step, init, (Wks, alphas, tends, W0s, b0s))

    ng = _N // _IN_SZ
    hm = out.reshape(ng, _IN_SZ * _H)
    return _tc_final(hm, Wl, bl.reshape(1, _OUT))


# speedup vs baseline: 2.0613x; 2.0613x over previous
import jax
import jax.numpy as jnp
from jax import lax
from jax.experimental import pallas as pl
from jax.experimental.pallas import tpu as pltpu
from jax.experimental.pallas import tpu_sc as plsc

_N = 100000
_E = 1600000
_H = 32
_HH = 16
_K = 5
_IN_SZ = 1000
_OUT = 10

_NSC = 2
_NTILE = 16
_HALF = _N // _NSC
_G = 128
_NGRP = 12800
_EP = _NGRP * _G
_EPB = _E + 8192
_GPT = _NGRP // _NTILE
_B = 2
_BE = _B * _G
_BNC = 104
_ROWS_PT = 3128
_SROWS = _NTILE * _ROWS_PT
_DUMP = _HALF + 5

_R = 2000
_NBLK = _N // _R


def _make_sc_prop(width, histogram, nslab=1):
    if width == 1:
        rows_s, acc_s, bnc_s, out_s = (_BE,), (_SROWS,), (_BNC,), (_N,)
    else:
        rows_s = (_BE, width)
        acc_s = (_SROWS, width)
        bnc_s = (_BNC, width)
        out_s = (_N, width)
    mesh = plsc.VectorSubcoreMesh(core_axis_name="c", subcore_axis_name="s")
    scratch = [
        pltpu.VMEM((16,), jnp.int32),
        pltpu.VMEM((_BE,), jnp.int32),
        pltpu.VMEM((_BE,), jnp.int32),
        pltpu.VMEM((_BE,), jnp.int32),
        pltpu.VMEM((_BE,), jnp.int32),
        pltpu.VMEM(rows_s, jnp.float32),
        pltpu.VMEM(rows_s, jnp.float32),
        pltpu.VMEM(bnc_s, jnp.float32),
        pltpu.VMEM_SHARED(acc_s, jnp.float32),
        pltpu.SemaphoreType.DMA,
        pltpu.SemaphoreType.DMA,
        pltpu.SemaphoreType.DMA,
        pltpu.SemaphoreType.DMA,
    ]

    def body(gidx_hbm, sidx_hbm, gb_hbm, *rest):
        w_hbms = rest[:nslab]
        zeros_hbm = rest[nslab]
        out_hbms = rest[nslab + 1:2 * nslab + 1]
        (gbv, rowi, coli, tgti0, tgti1, rows0, rows1, wb, accum,
         sg0, sg1, ss0, ss1) = rest[2 * nslab + 1:]
        c = lax.axis_index("c")
        s = lax.axis_index("s")
        base = c * _HALF
        lo = s * _ROWS_PT
        tail = _HALF - (_NTILE - 1) * _ROWS_PT

        pltpu.sync_copy(zeros_hbm, wb)
        if histogram:
            pltpu.sync_copy(w_hbms[0], rows0)
        else:
            pltpu.sync_copy(gb_hbm, gbv)


        def _chunks(total):
            offs = []
            o = 0
            while o < total:
                offs.append((o, min(_BNC, total - o)))
                o += _BNC
            return offs

        for slab in range(nslab):
            w_hbm = w_hbms[slab]
            out_hbm = out_hbms[slab]

            for off, sz in _chunks(_ROWS_PT):
                pltpu.sync_copy(wb.at[pl.ds(0, sz)],
                                accum.at[pl.ds(lo + off, sz)])
            plsc.subcore_barrier()

            nb = _GPT // _B

            def _compute_tgt(e0, tgti, emax):
                pltpu.sync_copy(sidx_hbm.at[pl.ds(e0, _BE)], coli)
                lane16 = lax.iota(jnp.int32, 16)
                for j in range(_BE // 16):
                    v = coli[pl.ds(j * 16, 16)]
                    t0 = v - base
                    ok = (t0 >= 0) & (t0 < _HALF)
                    ok = ok & ((e0 + j * 16 + lane16) < emax)
                    tgti[pl.ds(j * 16, 16)] = jnp.where(ok, t0, _DUMP)

            if histogram:
                def hstep(bt, carry):
                    e0 = (s * _GPT + bt * _B) * _G
                    _compute_tgt(e0, tgti0, _EP)
                    pltpu.sync_copy(rows0, accum.at[tgti0], add=True)
                    return carry

                lax.fori_loop(0, nb, hstep, 0)
            else:
                gbvec = gbv[pl.ds(0, 16)]
                lane = lax.iota(jnp.int32, 16)
                gstart = jnp.sum(jnp.where(lane == 2 * c, gbvec, 0))
                gend = jnp.sum(jnp.where(lane == 2 * c + 1, gbvec, 0))
                gpt_t = lax.div(gend - gstart + _NTILE - 1, _NTILE)
                tstart = gstart + s * gpt_t
                emax = jnp.minimum(tstart + gpt_t, gend) * _G
                nsup = jnp.maximum(
                    lax.div(gpt_t + 2 * _B - 1, 2 * _B), 1)
                bufs = ((tgti0, rows0, sg0, ss0), (tgti1, rows1, sg1, ss1))

                def _half(i, k, par):
                    tg, rw, sg, ss = bufs[par]
                    tgp, rwp, sgp, ssp = bufs[1 - par]
                    e0 = (tstart + i * _B) * _G

                    @pl.when(k > 0)
                    def _():
                        pltpu.make_async_copy(
                            rw, accum.at[tg], ss).wait()

                    _compute_tgt(e0, tg, emax)

                    @pl.when(i > 0)
                    def _():
                        pltpu.make_async_copy(
                            w_hbm.at[rowi], rwp, sgp).wait()
                        pltpu.async_copy(
                            rwp, accum.at[tgp], ssp, add=True)

                    pltpu.sync_copy(gidx_hbm.at[pl.ds(e0, _BE)], rowi)
                    pltpu.async_copy(w_hbm.at[rowi], rw, sg)

                def pstep(k, carry):
                    _half(2 * k, k, 0)
                    _half(2 * k + 1, k, 1)
                    return carry

                lax.fori_loop(0, nsup, pstep, 0)
                tg, rw, sg, ss = bufs[1]
                tgp, rwp, sgp, ssp = bufs[0]
                pltpu.make_async_copy(w_hbm.at[rowi], rw, sg).wait()
                pltpu.async_copy(rw, accum.at[tg], ss, add=True)
                pltpu.make_async_copy(rwp, accum.at[tgp], ssp).wait()
                pltpu.make_async_copy(rw, accum.at[tg], ss).wait()
            plsc.subcore_barrier()

            @pl.when(s < _NTILE - 1)
            def _():
                for off, sz in _chunks(_ROWS_PT):
                    pltpu.sync_copy(accum.at[pl.ds(lo + off, sz)],
                                    wb.at[pl.ds(0, sz)])
                    pltpu.sync_copy(wb.at[pl.ds(0, sz)],
                                    out_hbm.at[pl.ds(base + lo + off, sz)])

            @pl.when(s == _NTILE - 1)
            def _():
                for off, sz in _chunks(tail):
                    pltpu.sync_copy(accum.at[pl.ds(lo + off, sz)],
                                    wb.at[pl.ds(0, sz)])
                    pltpu.sync_copy(wb.at[pl.ds(0, sz)],
                                    out_hbm.at[pl.ds(base + lo + off, sz)])

            if slab + 1 < nslab:
                pltpu.sync_copy(zeros_hbm, wb)

    if nslab == 1:
        out_type = jax.ShapeDtypeStruct(out_s, jnp.float32)
    else:
        out_type = [jax.ShapeDtypeStruct(out_s, jnp.float32)] * 2
    return pl.kernel(
        body,
        out_type=out_type,
        mesh=mesh,
        scratch_types=scratch,
        compiler_params=pltpu.CompilerParams(use_tc_tiling_on_sc=False,
                                             needs_layout_passes=False),
    )


def _row_spec(w):
    return pl.BlockSpec((_R, w), lambda i: (i, 0))


def _full_spec(shape):
    return pl.BlockSpec(shape, lambda i: tuple(0 for _ in shape))


def _tc_rsqrt(deg):
    def body(d_ref, o_ref):
        d = d_ref[...]
        o_ref[...] = jnp.where(d > 0, lax.rsqrt(jnp.where(d > 0, d, 1.0)), 0.0)

    return pl.pallas_call(
        body,
        grid=(_NBLK,),
        in_specs=[_row_spec(1)],
        out_specs=_row_spec(1),
        out_shape=jax.ShapeDtypeStruct((_N, 1), jnp.float32),
    )(deg)


def _tc_init(x, dis2, W1r, b):
    def body(x_ref, d_ref, w_ref, b_ref, out_ref, h_ref, wout_ref):
        xv = x_ref[...]
        out_ref[...] = xv * w_ref[...] + b_ref[...]
        hv = jnp.broadcast_to(xv, (_R, _H))
        h_ref[...] = hv
        wout_ref[...] = d_ref[...] * hv

    return pl.pallas_call(
        body,
        grid=(_NBLK,),
        in_specs=[_row_spec(1), _row_spec(1), _full_spec((1, _H)),
                  _full_spec((1, _H))],
        out_specs=[_row_spec(_H)] * 3,
        out_shape=[jax.ShapeDtypeStruct((_N, _H), jnp.float32)] * 3,
    )(x, dis2, W1r, b)


def _tc_step(acc, prev2, prev1, out_in, dis2, Wk, alpha, tend, W0n, b0n):
    def body(a_ref, p2_ref, p1_ref, out_ref, d_ref, w_ref, al_ref, te_ref,
             w0_ref, b0_ref, np2_ref, np1_ref, nw_ref, nout_ref):
        d = d_ref[...]
        tx = al_ref[0, 0] * (d * a_ref[...]) - p2_ref[...]
        o1 = out_ref[...] + jnp.dot(tx, w_ref[...],
                                    preferred_element_type=jnp.float32)
        te = te_ref[0, 0]
        h2 = jnp.maximum(o1, 0.0)
        o2 = jnp.dot(h2, w0_ref[...],
                     preferred_element_type=jnp.float32) + b0_ref[...]
        np2_ref[...] = (1.0 - te) * p1_ref[...]
        np1 = te * h2 + (1.0 - te) * tx
        np1_ref[...] = np1
        nw_ref[...] = d * np1
        nout_ref[...] = te * o2 + (1.0 - te) * o1

    return pl.pallas_call(
        body,
        grid=(_NBLK,),
        in_specs=[_row_spec(_H), _row_spec(_H), _row_spec(_H), _row_spec(_H),
                  _row_spec(1), _full_spec((_H, _H)), _full_spec((1, 1)),
                  _full_spec((1, 1)), _full_spec((_H, _H)),
                  _full_spec((1, _H))],
        out_specs=[_row_spec(_H)] * 4,
        out_shape=[jax.ShapeDtypeStruct((_N, _H), jnp.float32)] * 4,
    )(acc, prev2, prev1, out_in, dis2, Wk, alpha, tend, W0n, b0n)


def _tc_final(hm, Wl, bl2):
    kb = 3200
    nk = (_IN_SZ * _H) // kb
    ng = _N // _IN_SZ

    def body(h_ref, w_ref, b_ref, o_ref):
        @pl.when(pl.program_id(0) == 0)
        def _():
            o_ref[...] = jnp.zeros((ng, _OUT), jnp.float32) + b_ref[...]

        o_ref[...] += jnp.dot(h_ref[...], w_ref[...],
                              preferred_element_type=jnp.float32)

    return pl.pallas_call(
        body,
        grid=(nk,),
        in_specs=[pl.BlockSpec((ng, kb), lambda i: (0, i)),
                  pl.BlockSpec((kb, _OUT), lambda i: (i, 0)),
                  pl.BlockSpec((1, _OUT), lambda i: (0, 0))],
        out_specs=pl.BlockSpec((ng, _OUT), lambda i: (0, 0)),
        out_shape=jax.ShapeDtypeStruct((ng, _OUT), jnp.float32),
    )(hm, Wl, bl2)


_sc_hist = _make_sc_prop(1, histogram=True)
_sc_prop = _make_sc_prop(_H, histogram=False)


def kernel(x, edge_index, batch, W1, b1, W2, b2, W3, b3, Wl, bl):
    row = edge_index[0]
    col = edge_index[1]
    pad = _EP - _E
    rowg = jnp.concatenate([row, jnp.zeros((pad,), jnp.int32)])
    rowh = jnp.concatenate([row, jnp.full((pad,), _N, jnp.int32)])
    perm = jnp.argsort((col >= _HALF).astype(jnp.int32), stable=True)
    padb = _EPB - _E
    rowb = jnp.concatenate([row[perm], jnp.zeros((padb,), jnp.int32)])
    colb = jnp.concatenate([col[perm], jnp.full((padb,), _N, jnp.int32)])
    c0 = jnp.sum((col < _HALF).astype(jnp.int32))
    gb = (jnp.zeros((16,), jnp.int32)
          .at[1].set((c0 + _G - 1) // _G)
          .at[2].set(c0 // _G)
          .at[3].set(_E // _G))
    z1 = jnp.zeros((_BNC,), jnp.float32)
    zH = jnp.zeros((_BNC, _H), jnp.float32)
    onesg = jnp.ones((_BE,), jnp.float32)

    deg = _sc_hist(rowg, rowh, gb, onesg, z1)
    dis2 = _tc_rsqrt(deg.reshape(_N, 1))

    def padW1(k):
        return jnp.zeros((_H, _H), jnp.float32).at[0].set(W1[k, 0])

    Wks = jnp.stack([padW1(1), padW1(2), padW1(3), padW1(4),
                     W2[1], W2[2], W2[3], W2[4],
                     W3[1], W3[2], W3[3], W3[4]])
    alphas = jnp.tile(jnp.array([-1.0, -2.0, -2.0, -2.0], jnp.float32),
                      3).reshape(12, 1, 1)
    tends = jnp.array([0, 0, 0, 1, 0, 0, 0, 1, 0, 0, 0, 0],
                      jnp.float32).reshape(12, 1, 1)
    zW0 = jnp.zeros((_H, _H), jnp.float32)
    zb0 = jnp.zeros((1, _H), jnp.float32)
    W0s = jnp.stack([zW0, zW0, zW0, W2[0], zW0, zW0, zW0, W3[0],
                     zW0, zW0, zW0, zW0])
    b0s = jnp.stack([zb0, zb0, zb0, b2.reshape(1, _H), zb0, zb0, zb0,
                     b3.reshape(1, _H), zb0, zb0, zb0, zb0])

    out0, h0, w0 = _tc_init(x, dis2, W1[0].reshape(1, _H),
                            b1.reshape(1, _H))

    def step(carry, xs):
        prev2, prev1, w, out = carry
        Wk, alpha, tend, W0n, b0n = xs
        acc = _sc_prop(rowb, colb, gb, w, zH)
        np2, np1, nw, nout = _tc_step(acc, prev2, prev1, out, dis2,
                                      Wk, alpha, tend, W0n, b0n)
        return (np2, np1, nw, nout), 0.0

    init = (jnp.zeros((_N, _H), jnp.float32), h0, w0, out0)
    (_, _, _, out), _ = lax.scan(step, init, (Wks, alphas, tends, W0s, b0s))

    ng = _N // _IN_SZ
    hm = out.reshape(ng, _IN_SZ * _H)
    return _tc_final(hm, Wl, bl.reshape(1, _OUT))
